# baseline (speedup 1.0000x reference)
import functools

import jax
import jax.numpy as jnp
from jax import lax
from jax.experimental import pallas as pl
from jax.experimental.pallas import tpu as pltpu
from jax.experimental.pallas import tpu_sc as plsc

_TOTAL = 8192
_DIM = 64
_NW = 32
_B = _TOTAL // _NW
_L = 16


def _rsqrt(x):
    xi = plsc.bitcast(x, jnp.int32)
    yi = jnp.int32(0x5F3759DF) - (xi >> 1)
    y = plsc.bitcast(yi, jnp.float32)
    for _ in range(3):
        y = y * (1.5 - 0.5 * x * y * y)
    return y


@jax.jit
def _scores(h, r, t, ent_emb, rel_emb):
    @functools.partial(
        pl.kernel,
        mesh=plsc.VectorSubcoreMesh(core_axis_name="c", subcore_axis_name="s"),
        out_type=jax.ShapeDtypeStruct((_TOTAL,), jnp.float32),
        compiler_params=pltpu.CompilerParams(
            needs_layout_passes=False, use_tc_tiling_on_sc=False),
        scratch_types=[
            pltpu.VMEM((_B,), jnp.int32),
            pltpu.VMEM((_B,), jnp.int32),
            pltpu.VMEM((_B,), jnp.int32),
            pltpu.VMEM((_B, _DIM), jnp.float32),
            pltpu.VMEM((_B, _DIM), jnp.float32),
            pltpu.VMEM((_B, _DIM), jnp.float32),
            pltpu.VMEM((_B,), jnp.float32),
            pltpu.SemaphoreType.DMA,
        ],
    )
    def k(h_hbm, r_hbm, t_hbm, ent_hbm, rel_hbm, out_hbm,
          hi, ri, ti, hrow, rrow, trow, sc, sem):
        wid = lax.axis_index("s") * 2 + lax.axis_index("c")
        base = wid * _B
        pltpu.sync_copy(h_hbm.at[pl.ds(base, _B)], hi)
        pltpu.sync_copy(r_hbm.at[pl.ds(base, _B)], ri)
        pltpu.sync_copy(t_hbm.at[pl.ds(base, _B)], ti)
        cph = pltpu.async_copy(ent_hbm.at[hi], hrow, sem)
        cpr = pltpu.async_copy(rel_hbm.at[ri], rrow, sem)
        cpt = pltpu.async_copy(ent_hbm.at[ti], trow, sem)
        cph.wait()
        cpr.wait()
        cpt.wait()

        lanes = lax.iota(jnp.int32, _L)
        zero = jnp.zeros((_L,), jnp.float32)

        def grp(g, carry):
            rows = g * _L + lanes
            vhh = vrr = vtt = vhr = vht = vrt = zero
            for c in range(_DIM):
                col = jnp.full((_L,), c, jnp.int32)
                hc = plsc.load_gather(hrow, [rows, col])
                rc = plsc.load_gather(rrow, [rows, col])
                tc = plsc.load_gather(trow, [rows, col])
                vhh = vhh + hc * hc
                vrr = vrr + rc * rc
                vtt = vtt + tc * tc
                vhr = vhr + hc * rc
                vht = vht + hc * tc
                vrt = vrt + rc * tc
            s2 = 3.0 + 2.0 * (vhr * _rsqrt(vhh * vrr)
                              - vht * _rsqrt(vhh * vtt)
                              - vrt * _rsqrt(vrr * vtt))
            s2 = jnp.maximum(s2, 0.0)
            sc[pl.ds(g * _L, _L)] = s2 * _rsqrt(jnp.maximum(s2, 1e-20))
            return carry

        lax.fori_loop(0, _B // _L, grp, 0)
        pltpu.sync_copy(sc, out_hbm.at[pl.ds(base, _B)])

    return k(h, r, t, ent_emb, rel_emb)


def kernel(h, r, t, ent_emb, rel_emb):
    h = h.astype(jnp.int32)
    r = r.astype(jnp.int32)
    t = t.astype(jnp.int32)
    score = _scores(h, r, t, ent_emb, rel_emb)
    bs = _TOTAL // 2
    p_score = score[:bs].reshape(1, bs).transpose(1, 0)
    n_score = score[bs:].reshape(1, bs).transpose(1, 0)
    return (p_score, n_score)
